# Initial kernel scaffold; baseline (speedup 1.0000x reference)
import jax
import jax.numpy as jnp
from jax.experimental import pallas as pl


def kernel(x, W):
    raise NotImplementedError("write your pallas kernel here")


# speedup vs baseline: 28.9301x; 28.9301x over previous
import functools

import jax
import jax.numpy as jnp
from jax import lax
from jax.experimental import pallas as pl
from jax.experimental.pallas import tpu as pltpu
from jax.experimental.pallas import tpu_sc as plsc

_FIELD_IDX = (0,) * 7 + (1,) * 7 + (2,) * 6 + (3,) * 6
_NF = 26
_NT = 4
_D = 16
_B = 4096
_ROWS = 260000
_PAIRS = _NF * (_NF - 1) // 2
_LOOK = _NT * _NF

_NC = 2
_NS = 16
_NW = _NC * _NS
_PER_W = _B * _LOOK // _NW
_IDXROWS = _PER_W // 128
_CHUNKS = 8
_CROWS = _PER_W // _CHUNKS
_G = _CROWS // 128


def _sc_gather_body(w_hbm, gidx_hbm, out_hbm, idx_v, rows_v, sem):
    wid = lax.axis_index("s") * _NC + lax.axis_index("c")
    pltpu.sync_copy(gidx_hbm.at[pl.ds(wid * _IDXROWS, _IDXROWS)], idx_v)
    for c in range(_CHUNKS):
        cps = [
            pltpu.async_copy(
                w_hbm.at[idx_v.at[c * _G + g]],
                rows_v.at[pl.ds(g * 128, 128)],
                sem,
            )
            for g in range(_G)
        ]
        for cp in cps:
            cp.wait()
        pltpu.sync_copy(
            rows_v, out_hbm.at[pl.ds(wid * _PER_W + c * _CROWS, _CROWS)]
        )


@functools.cache
def _sc_gather():
    return functools.partial(
        pl.kernel,
        mesh=plsc.VectorSubcoreMesh(core_axis_name="c", subcore_axis_name="s"),
        out_type=jax.ShapeDtypeStruct((_B * _LOOK, _D), jnp.float32),
        scratch_types=[
            pltpu.VMEM((_IDXROWS, 128), jnp.int32),
            pltpu.VMEM((_CROWS, _D), jnp.float32),
            pltpu.SemaphoreType.DMA,
        ],
        compiler_params=pltpu.CompilerParams(use_tc_tiling_on_sc=False),
    )(_sc_gather_body)


_SEGS = []
for _i in range(_NF - 1):
    _segs = []
    _j = _i + 1
    while _j < _NF:
        _f = _FIELD_IDX[_j]
        _j2 = _j
        while _j2 < _NF and _FIELD_IDX[_j2] == _f:
            _j2 += 1
        _segs.append((_f, _j2 - _j))
        _j = _j2
    _SEGS.append(_segs)


def _pairs_body(e_ref, o_ref):
    e = e_ref[...]
    parts = []
    for i in range(_NF - 1):
        fi = _FIELD_IDX[i]
        r = e[:, (fi * _NF + i + 1) * _D:(fi * _NF + _NF) * _D]
        lsegs = []
        for f, cnt in _SEGS[i]:
            v = e[:, (f * _NF + i) * _D:(f * _NF + i + 1) * _D]
            lsegs.append(v if cnt == 1 else jnp.tile(v, (1, cnt)))
        left = lsegs[0] if len(lsegs) == 1 else jnp.concatenate(lsegs, axis=1)
        parts.append(left * r)
    o_ref[...] = jnp.concatenate(parts, axis=1)


_BB = 256
_TC_PAIRS = pl.pallas_call(
    _pairs_body,
    grid=(_B // _BB,),
    in_specs=[pl.BlockSpec((_BB, _LOOK * _D), lambda i: (i, 0))],
    out_specs=pl.BlockSpec((_BB, _PAIRS * _D), lambda i: (i, 0)),
    out_shape=jax.ShapeDtypeStruct((_B, _PAIRS * _D), jnp.float32),
    compiler_params=pltpu.CompilerParams(dimension_semantics=("arbitrary",)),
)


def kernel(x, W):
    wf = W.reshape(_NT * _ROWS, _D)
    feat_offs = (jnp.arange(_NF, dtype=x.dtype) * 10000)[None, None, :]
    table_offs = (jnp.arange(_NT, dtype=x.dtype) * _ROWS)[None, :, None]
    gidx = (x[:, None, :] + feat_offs + table_offs).reshape(_IDXROWS * _NW, 128)
    e = _sc_gather()(wf, gidx)
    out = _TC_PAIRS(e.reshape(_B, _LOOK * _D))
    return out.reshape(_B, _PAIRS, _D)
